# baseline (speedup 1.0000x reference)
import functools

import jax
import jax.numpy as jnp
from jax.experimental import pallas as pl
from jax.experimental.pallas import tpu as pltpu

_EPS = 1e-5


def _fused_mlp_kernel(x_ref, W1_ref, g1_ref, be1_ref, W2_ref, g2_ref, be2_ref,
                      Wf1_ref, bf1_ref, Wf2_ref, bf2_ref, out_ref,
                      h_scr, S_scr, cs_ref, s_ref, q_ref, sh_ref,
                      W1s_scr, Wf1s_scr, c_ref,
                      *, n_rows, bm):
    p = pl.program_id(0)
    i = pl.program_id(1)
    rows = pl.ds(i * bm, bm)
    inv_n = 1.0 / n_rows

    @pl.when(p == 0)
    def _phase0():
        @pl.when(i == 0)
        def _init():
            S_scr[...] = jnp.zeros_like(S_scr)
            cs_ref[...] = jnp.zeros_like(cs_ref)
            s_ref[...] = jnp.zeros_like(s_ref)
            q_ref[...] = jnp.zeros_like(q_ref)

        xb = x_ref[...]
        S_scr[...] += jax.lax.dot_general(
            xb, xb, (((0,), (0,)), ((), ())),
            preferred_element_type=jnp.float32)
        cs_ref[...] += jnp.sum(xb, axis=0, keepdims=True)

    @pl.when(p == 1)
    def _phase1():
        @pl.when(i == 0)
        def _finalize_bn1():
            mean = jnp.dot(cs_ref[...], W1_ref[...],
                           preferred_element_type=jnp.float32) * inv_n
            T = jnp.dot(S_scr[...], W1_ref[...],
                        preferred_element_type=jnp.float32)
            m2 = jnp.sum(W1_ref[...] * T, axis=0, keepdims=True) * inv_n
            var = m2 - mean * mean
            scale = g1_ref[...] * jax.lax.rsqrt(var + _EPS)
            W1s_scr[...] = W1_ref[...] * scale
            sh_ref[...] = be1_ref[...] - mean * scale

        u = jnp.dot(x_ref[...], W1s_scr[...],
                    preferred_element_type=jnp.float32)
        h1 = jnp.maximum(u + sh_ref[...], 0.0)
        h2 = jnp.dot(h1, W2_ref[...], preferred_element_type=jnp.float32)
        h_scr[rows, :] = h2
        s_ref[...] += jnp.sum(h2, axis=0, keepdims=True)
        q_ref[...] += jnp.sum(h2 * h2, axis=0, keepdims=True)

    @pl.when(p == 2)
    def _phase2():
        @pl.when(i == 0)
        def _finalize_bn2():
            mean = s_ref[...] * inv_n
            var = q_ref[...] * inv_n - mean * mean
            scale = g2_ref[...] * jax.lax.rsqrt(var + _EPS)
            shift = be2_ref[...] - mean * scale
            Wf1s_scr[...] = Wf1_ref[...] * scale.reshape(-1, 1)
            c_ref[...] = jnp.dot(shift, Wf1_ref[...],
                                 preferred_element_type=jnp.float32) + bf1_ref[...]

        m = jnp.dot(h_scr[rows, :], Wf1s_scr[...],
                    preferred_element_type=jnp.float32)
        m = jnp.maximum(m + c_ref[...], 0.0)
        out_ref[...] = jnp.dot(m, Wf2_ref[...],
                               preferred_element_type=jnp.float32) + bf2_ref[...]


def kernel(x, edge_index, edge_attr, W1, b1, g1, be1, W2, b2, g2, be2,
           Wf1, bf1, Wf2, bf2):
    del edge_index, edge_attr, b1, b2
    n, f_in = x.shape
    h_dim = W1.shape[1]
    mid = Wf1.shape[1]
    out_c = Wf2.shape[1]

    bm = 1000
    nb = n // bm

    full = lambda shape: pl.BlockSpec(shape, lambda p, i: (0, 0))
    row2 = lambda f: (1, f)

    grid = (3, nb)
    body = functools.partial(_fused_mlp_kernel, n_rows=n, bm=bm)
    out = pl.pallas_call(
        body,
        grid=grid,
        in_specs=[
            pl.BlockSpec((bm, f_in), lambda p, i: (jnp.where(p < 2, i, 0), 0)),
            full((f_in, h_dim)),
            full(row2(h_dim)),
            full(row2(h_dim)),
            full((h_dim, h_dim)),
            full(row2(h_dim)),
            full(row2(h_dim)),
            full((h_dim, mid)),
            full(row2(mid)),
            full((mid, out_c)),
            full(row2(out_c)),
        ],
        out_specs=pl.BlockSpec((bm, out_c), lambda p, i: (i, 0)),
        out_shape=jax.ShapeDtypeStruct((n, out_c), jnp.float32),
        scratch_shapes=[
            pltpu.VMEM((n, h_dim), jnp.float32),
            pltpu.VMEM((f_in, f_in), jnp.float32),
            pltpu.VMEM((1, f_in), jnp.float32),
            pltpu.VMEM((1, h_dim), jnp.float32),
            pltpu.VMEM((1, h_dim), jnp.float32),
            pltpu.VMEM((1, h_dim), jnp.float32),
            pltpu.VMEM((f_in, h_dim), jnp.float32),
            pltpu.VMEM((h_dim, mid), jnp.float32),
            pltpu.VMEM((1, mid), jnp.float32),
        ],
        compiler_params=pltpu.CompilerParams(
            dimension_semantics=("arbitrary", "arbitrary"),
        ),
    )(
        x, W1, g1.reshape(1, -1), be1.reshape(1, -1),
        W2, g2.reshape(1, -1), be2.reshape(1, -1),
        Wf1, bf1.reshape(1, -1), Wf2, bf2.reshape(1, -1),
    )
    return out


# speedup vs baseline: 1.3319x; 1.3319x over previous
import functools

import jax
import jax.numpy as jnp
from jax.experimental import pallas as pl
from jax.experimental.pallas import tpu as pltpu

_EPS = 1e-5


def _fused_mlp_kernel(x_ref, W1_ref, g1_ref, be1_ref, W2_ref, g2_ref, be2_ref,
                      Wf1_ref, bf1_ref, Wf2_ref, bf2_ref, out_ref, h_scr,
                      *, n_rows, bm):
    nchunks = n_rows // bm
    inv_n = 1.0 / n_rows
    W1 = W1_ref[...]

    x = x_ref[...]
    S = jax.lax.dot_general(x, x, (((0,), (0,)), ((), ())),
                            preferred_element_type=jnp.float32)
    cs = jnp.sum(x, axis=0, keepdims=True)
    mean1 = jnp.dot(cs, W1, preferred_element_type=jnp.float32) * inv_n
    T = jnp.dot(S, W1, preferred_element_type=jnp.float32)
    m2 = jnp.sum(W1 * T, axis=0, keepdims=True) * inv_n
    var1 = m2 - mean1 * mean1
    sc1 = g1_ref[...] * jax.lax.rsqrt(var1 + _EPS)
    sh1 = be1_ref[...] - mean1 * sc1
    W1s = W1 * sc1

    W2 = W2_ref[...]
    s = jnp.zeros((1, W2.shape[1]), jnp.float32)
    q = jnp.zeros((1, W2.shape[1]), jnp.float32)
    for k in range(nchunks):
        rows = pl.ds(k * bm, bm)
        u = jnp.dot(x_ref[rows, :], W1s, preferred_element_type=jnp.float32)
        h1 = jnp.maximum(u + sh1, 0.0)
        h2 = jnp.dot(h1, W2, preferred_element_type=jnp.float32)
        h_scr[rows, :] = h2
        s = s + jnp.sum(h2, axis=0, keepdims=True)
        q = q + jnp.sum(h2 * h2, axis=0, keepdims=True)

    mean2 = s * inv_n
    var2 = q * inv_n - mean2 * mean2
    sc2 = g2_ref[...] * jax.lax.rsqrt(var2 + _EPS)
    sh2 = be2_ref[...] - mean2 * sc2
    Wf1s = Wf1_ref[...] * sc2.reshape(-1, 1)
    c = jnp.dot(sh2, Wf1_ref[...],
                preferred_element_type=jnp.float32) + bf1_ref[...]

    Wf2 = Wf2_ref[...]
    bf2 = bf2_ref[...]
    for k in range(nchunks):
        rows = pl.ds(k * bm, bm)
        m = jnp.dot(h_scr[rows, :], Wf1s, preferred_element_type=jnp.float32)
        m = jnp.maximum(m + c, 0.0)
        out_ref[rows, :] = jnp.dot(m, Wf2,
                                   preferred_element_type=jnp.float32) + bf2


def kernel(x, edge_index, edge_attr, W1, b1, g1, be1, W2, b2, g2, be2,
           Wf1, bf1, Wf2, bf2):
    del edge_index, edge_attr, b1, b2
    n, f_in = x.shape
    h_dim = W1.shape[1]
    out_c = Wf2.shape[1]

    body = functools.partial(_fused_mlp_kernel, n_rows=n, bm=2000)
    out = pl.pallas_call(
        body,
        out_shape=jax.ShapeDtypeStruct((n, out_c), jnp.float32),
        scratch_shapes=[
            pltpu.VMEM((n, h_dim), jnp.float32),
        ],
    )(
        x, W1, g1.reshape(1, -1), be1.reshape(1, -1),
        W2, g2.reshape(1, -1), be2.reshape(1, -1),
        Wf1, bf1.reshape(1, -1), Wf2, bf2.reshape(1, -1),
    )
    return out
